# Initial kernel scaffold; baseline (speedup 1.0000x reference)
import jax
import jax.numpy as jnp
from jax.experimental import pallas as pl


def kernel(feature, condition, edge_index, W_enc_i2h, b_enc_i2h, W_mean, b_mean, W_logvar, b_logvar, W_dec_i2h, b_dec_i2h, W_dec_out, b_dec_out):
    raise NotImplementedError("write your pallas kernel here")


# speedup vs baseline: 6.9119x; 6.9119x over previous
import functools

import jax
import jax.numpy as jnp
from jax import lax
from jax.experimental import pallas as pl
from jax.experimental.pallas import tpu as pltpu
from jax.experimental.pallas import tpu_sc as plsc

N = 10000
E = 320000
FEAT = 128
COND = 16
HID = 128
LAT = 64

NC = 2
NS = 16
NW = NC * NS
CHUNK = 128
NCH = 80
GRP = 8
NGRP = NCH // GRP
E_W = CHUNK * NCH
E_PAD = E_W * NW
PAD_DST = N
ACC_ROWS = 10112
ZROWS = ACC_ROWS // NS

_mesh = plsc.VectorSubcoreMesh(core_axis_name="c", subcore_axis_name="s")


@functools.partial(
    pl.kernel,
    out_type=jax.ShapeDtypeStruct((NC, ACC_ROWS, HID), jnp.float32),
    mesh=_mesh,
    scratch_types=[
        pltpu.VMEM((GRP, CHUNK), jnp.int32),
        pltpu.VMEM((CHUNK, HID), jnp.float32),
        pltpu.VMEM((64, HID), jnp.float32),
        pltpu.VMEM_SHARED((ACC_ROWS, HID), jnp.float32),
    ],
)
def _sc_degree(dst_hbm, deg_hbm, idx_g, ones_v, zbuf, deg_sh):
    c = lax.axis_index("c")
    s = lax.axis_index("s")
    wid = c * NS + s

    def fill(i, _):
        for k in range(HID // 16):
            ones_v[i, pl.ds(k * 16, 16)] = jnp.ones((16,), jnp.float32)
        return 0
    lax.fori_loop(0, CHUNK, fill, 0)

    def fillz(i, _):
        for k in range(HID // 16):
            zbuf[i, pl.ds(k * 16, 16)] = jnp.zeros((16,), jnp.float32)
        return 0
    lax.fori_loop(0, 64, fillz, 0)

    for k in range(ZROWS // 64):
        pltpu.sync_copy(zbuf, deg_sh.at[pl.ds(s * ZROWS + k * 64, 64)])
    pltpu.sync_copy(zbuf.at[pl.ds(0, ZROWS % 64)],
                    deg_sh.at[pl.ds(s * ZROWS + (ZROWS // 64) * 64,
                                    ZROWS % 64)])
    plsc.subcore_barrier()

    def body(g, _):
        pltpu.sync_copy(dst_hbm.at[wid, pl.ds(g * GRP, GRP)], idx_g)
        for k in range(GRP):
            pltpu.sync_copy(ones_v, deg_sh.at[idx_g.at[k]], add=True)
        return 0
    lax.fori_loop(0, NGRP, body, 0)

    plsc.subcore_barrier()
    pltpu.sync_copy(deg_sh.at[pl.ds(s * ZROWS, ZROWS)],
                    deg_hbm.at[c, pl.ds(s * ZROWS, ZROWS)])


@functools.partial(
    pl.kernel,
    out_type=jax.ShapeDtypeStruct((NC, ACC_ROWS, HID), jnp.float32),
    mesh=_mesh,
    scratch_types=[
        pltpu.VMEM((GRP, CHUNK), jnp.int32),
        pltpu.VMEM((GRP, CHUNK), jnp.int32),
        pltpu.VMEM((CHUNK, HID), jnp.float32),
        pltpu.VMEM((CHUNK, HID), jnp.float32),
        pltpu.VMEM((64, HID), jnp.float32),
        pltpu.VMEM_SHARED((ACC_ROWS, HID), jnp.float32),
        pltpu.SemaphoreType.DMA,
        pltpu.SemaphoreType.DMA,
    ],
)
def _sc_conv(hs_hbm, src_hbm, dst_hbm, acc_hbm,
             src_g, dst_g, rows_a, rows_b, zbuf, acc_sh, sem_a, sem_b):
    c = lax.axis_index("c")
    s = lax.axis_index("s")
    wid = c * NS + s

    def fillz(i, _):
        for k in range(HID // 16):
            zbuf[i, pl.ds(k * 16, 16)] = jnp.zeros((16,), jnp.float32)
        return 0
    lax.fori_loop(0, 64, fillz, 0)

    for k in range(ZROWS // 64):
        pltpu.sync_copy(zbuf, acc_sh.at[pl.ds(s * ZROWS + k * 64, 64)])
    pltpu.sync_copy(zbuf.at[pl.ds(0, ZROWS % 64)],
                    acc_sh.at[pl.ds(s * ZROWS + (ZROWS // 64) * 64,
                                    ZROWS % 64)])
    plsc.subcore_barrier()

    rows = (rows_a, rows_b)
    sems = (sem_a, sem_b)

    def group(g, _):
        pltpu.sync_copy(src_hbm.at[wid, pl.ds(g * GRP, GRP)], src_g)
        pltpu.sync_copy(dst_hbm.at[wid, pl.ds(g * GRP, GRP)], dst_g)
        pltpu.async_copy(hs_hbm.at[src_g.at[0]], rows_a, sem_a)
        for k in range(GRP):
            pltpu.make_async_copy(
                hs_hbm.at[src_g.at[k]], rows[k % 2], sems[k % 2]).wait()
            if k + 1 < GRP:
                pltpu.async_copy(
                    hs_hbm.at[src_g.at[k + 1]], rows[(k + 1) % 2],
                    sems[(k + 1) % 2])
            pltpu.sync_copy(rows[k % 2], acc_sh.at[dst_g.at[k]], add=True)
        return 0
    lax.fori_loop(0, NGRP, group, 0)

    plsc.subcore_barrier()
    pltpu.sync_copy(acc_sh.at[pl.ds(s * ZROWS, ZROWS)],
                    acc_hbm.at[c, pl.ds(s * ZROWS, ZROWS)])


BM = 2000
GRID = N // BM


def _row_spec(w):
    return pl.BlockSpec((BM, w), lambda i: (i, 0))


def _full_spec(shape):
    return pl.BlockSpec(shape, lambda i: tuple(0 for _ in shape))


_acc_spec = pl.BlockSpec((NC, BM, HID), lambda i: (0, i, 0))
_deg_spec = pl.BlockSpec((NC, BM, HID), lambda i: (0, i, 0))


def _dinv(deg_block):
    deg = deg_block[0, :, 0:1] + deg_block[1, :, 0:1] + 1.0
    return lax.rsqrt(deg)


def _tc_enc_body(feat_ref, cond_ref, w1f_ref, w1c_ref, deg_ref, hs1_ref):
    dinv = _dinv(deg_ref[...])
    h = jnp.dot(feat_ref[...], w1f_ref[...], preferred_element_type=jnp.float32)
    h = h + jnp.dot(cond_ref[...], w1c_ref[...], preferred_element_type=jnp.float32)
    hs1_ref[...] = h * dinv


def _tc_mid1_body(acc_ref, hs1_ref, b1_ref, wml_ref, deg_ref, hs2_ref):
    dinv = _dinv(deg_ref[...])
    a = acc_ref[...]
    h = (a[0] + a[1] + hs1_ref[...]) * dinv + b1_ref[...]
    h = jnp.maximum(h, 0.0)
    hs2_ref[...] = jnp.dot(h, wml_ref[...],
                           preferred_element_type=jnp.float32) * dinv


def _tc_mid2_body(acc_ref, hs2_ref, bml_ref, noise_ref, cond_ref,
                  w4z_ref, w4c_ref, deg_ref, ml_ref, z_ref, hs3_ref):
    dinv = _dinv(deg_ref[...])
    a = acc_ref[...]
    ml = (a[0] + a[1] + hs2_ref[...]) * dinv + bml_ref[...]
    mean = ml[:, :LAT]
    logvar = ml[:, LAT:]
    z = noise_ref[...] * jnp.exp(0.5 * logvar) + mean
    h2 = jnp.dot(z, w4z_ref[...], preferred_element_type=jnp.float32)
    h2 = h2 + jnp.dot(cond_ref[...], w4c_ref[...],
                      preferred_element_type=jnp.float32)
    ml_ref[...] = ml
    z_ref[...] = z
    hs3_ref[...] = h2 * dinv


def _tc_mid3_body(acc_ref, hs3_ref, b4_ref, w5_ref, deg_ref, hs4_ref):
    dinv = _dinv(deg_ref[...])
    a = acc_ref[...]
    h2 = (a[0] + a[1] + hs3_ref[...]) * dinv + b4_ref[...]
    h2 = jnp.maximum(h2, 0.0)
    hs4_ref[...] = jnp.dot(h2, w5_ref[...],
                           preferred_element_type=jnp.float32) * dinv


def _tc_out_body(acc_ref, hs4_ref, b5_ref, deg_ref, out_ref):
    dinv = _dinv(deg_ref[...])
    a = acc_ref[...]
    out_ref[...] = (a[0] + a[1] + hs4_ref[...]) * dinv + b5_ref[...]


def _f32(shape):
    return jax.ShapeDtypeStruct(shape, jnp.float32)


def kernel(feature, condition, edge_index,
           W_enc_i2h, b_enc_i2h, W_mean, b_mean, W_logvar, b_logvar,
           W_dec_i2h, b_dec_i2h, W_dec_out, b_dec_out):
    src = edge_index[0].astype(jnp.int32)
    dst = edge_index[1].astype(jnp.int32)
    pad = E_PAD - E
    src_p = jnp.concatenate([src, jnp.zeros((pad,), jnp.int32)])
    dst_p = jnp.concatenate([dst, jnp.full((pad,), PAD_DST, jnp.int32)])
    src_p = src_p.reshape(NW, NCH, CHUNK)
    dst_p = dst_p.reshape(NW, NCH, CHUNK)

    degacc = _sc_degree(dst_p)
    deg = degacc[:, :N, :]

    W1f = W_enc_i2h[:FEAT]
    W1c = W_enc_i2h[FEAT:]
    hs1 = pl.pallas_call(
        _tc_enc_body,
        grid=(GRID,),
        in_specs=[_row_spec(FEAT), _row_spec(COND),
                  _full_spec((FEAT, HID)), _full_spec((COND, HID)), _deg_spec],
        out_specs=_row_spec(HID),
        out_shape=_f32((N, HID)),
    )(feature, condition, W1f, W1c, deg)

    acc1 = _sc_conv(hs1, src_p, dst_p)[:, :N]

    Wml = jnp.concatenate([W_mean, W_logvar], axis=1)
    b1 = b_enc_i2h.reshape(1, HID)
    hs2 = pl.pallas_call(
        _tc_mid1_body,
        grid=(GRID,),
        in_specs=[_acc_spec, _row_spec(HID), _full_spec((1, HID)),
                  _full_spec((HID, HID)), _deg_spec],
        out_specs=_row_spec(HID),
        out_shape=_f32((N, HID)),
    )(acc1, hs1, b1, Wml, deg)

    acc2 = _sc_conv(hs2, src_p, dst_p)[:, :N]

    noise = jax.random.normal(jax.random.key(42), (N, LAT), jnp.float32)
    b_ml = jnp.concatenate([b_mean, b_logvar]).reshape(1, HID)
    W4z = W_dec_i2h[:LAT]
    W4c = W_dec_i2h[LAT:]
    ml, z, hs3 = pl.pallas_call(
        _tc_mid2_body,
        grid=(GRID,),
        in_specs=[_acc_spec, _row_spec(HID), _full_spec((1, HID)),
                  _row_spec(LAT), _row_spec(COND),
                  _full_spec((LAT, HID)), _full_spec((COND, HID)), _deg_spec],
        out_specs=[_row_spec(HID), _row_spec(LAT), _row_spec(HID)],
        out_shape=[_f32((N, HID)), _f32((N, LAT)), _f32((N, HID))],
    )(acc2, hs2, b_ml, noise, condition, W4z, W4c, deg)

    acc3 = _sc_conv(hs3, src_p, dst_p)[:, :N]

    b4 = b_dec_i2h.reshape(1, HID)
    hs4 = pl.pallas_call(
        _tc_mid3_body,
        grid=(GRID,),
        in_specs=[_acc_spec, _row_spec(HID), _full_spec((1, HID)),
                  _full_spec((HID, FEAT)), _deg_spec],
        out_specs=_row_spec(FEAT),
        out_shape=_f32((N, FEAT)),
    )(acc3, hs3, b4, W_dec_out, deg)

    acc4 = _sc_conv(hs4, src_p, dst_p)[:, :N]

    b5 = b_dec_out.reshape(1, FEAT)
    out = pl.pallas_call(
        _tc_out_body,
        grid=(GRID,),
        in_specs=[_acc_spec, _row_spec(FEAT), _full_spec((1, FEAT)), _deg_spec],
        out_specs=_row_spec(FEAT),
        out_shape=_f32((N, FEAT)),
    )(acc4, hs4, b5, deg)

    mean = ml[:, :LAT]
    logvar = ml[:, LAT:]
    return (z, mean, logvar, out)


# speedup vs baseline: 7.3450x; 1.0627x over previous
import functools

import jax
import jax.numpy as jnp
from jax import lax
from jax.experimental import pallas as pl
from jax.experimental.pallas import tpu as pltpu
from jax.experimental.pallas import tpu_sc as plsc

N = 10000
E = 320000
FEAT = 128
COND = 16
HID = 128
LAT = 64

NC = 2
NS = 16
NW = NC * NS
CHUNK = 64
GRP = 4
NGRP = 40
NB = 4
LAG = 3
E_W = CHUNK * GRP * NGRP
E_PAD = E_W * NW
TOT = GRP * NGRP
PAD_DST = N
ACC_ROWS = 10112
ZROWS = ACC_ROWS // NS

_mesh = plsc.VectorSubcoreMesh(core_axis_name="c", subcore_axis_name="s")


@functools.partial(
    pl.kernel,
    out_type=jax.ShapeDtypeStruct((NC, ACC_ROWS, HID), jnp.float32),
    mesh=_mesh,
    scratch_types=[
        pltpu.VMEM((GRP, CHUNK), jnp.int32),
        pltpu.VMEM((CHUNK, HID), jnp.float32),
        pltpu.VMEM((64, HID), jnp.float32),
        pltpu.VMEM_SHARED((ACC_ROWS, HID), jnp.float32),
    ],
)
def _sc_degree(dst_hbm, deg_hbm, idx_g, ones_v, zbuf, deg_sh):
    c = lax.axis_index("c")
    s = lax.axis_index("s")
    wid = c * NS + s

    def fill(i, _):
        for k in range(HID // 16):
            ones_v[i, pl.ds(k * 16, 16)] = jnp.ones((16,), jnp.float32)
        return 0
    lax.fori_loop(0, CHUNK, fill, 0)

    def fillz(i, _):
        for k in range(HID // 16):
            zbuf[i, pl.ds(k * 16, 16)] = jnp.zeros((16,), jnp.float32)
        return 0
    lax.fori_loop(0, 64, fillz, 0)

    for k in range(ZROWS // 64):
        pltpu.sync_copy(zbuf, deg_sh.at[pl.ds(s * ZROWS + k * 64, 64)])
    pltpu.sync_copy(zbuf.at[pl.ds(0, ZROWS % 64)],
                    deg_sh.at[pl.ds(s * ZROWS + (ZROWS // 64) * 64,
                                    ZROWS % 64)])
    plsc.subcore_barrier()

    def body(g, _):
        pltpu.sync_copy(dst_hbm.at[wid, g], idx_g)
        for k in range(GRP):
            pltpu.sync_copy(ones_v, deg_sh.at[idx_g.at[k]], add=True)
        return 0
    lax.fori_loop(0, NGRP, body, 0)

    plsc.subcore_barrier()
    pltpu.sync_copy(deg_sh.at[pl.ds(s * ZROWS, ZROWS)],
                    deg_hbm.at[c, pl.ds(s * ZROWS, ZROWS)])


@functools.partial(
    pl.kernel,
    out_type=jax.ShapeDtypeStruct((NC, ACC_ROWS, HID), jnp.float32),
    mesh=_mesh,
    scratch_types=[
        pltpu.VMEM((GRP, CHUNK), jnp.int32),
        pltpu.VMEM((GRP, CHUNK), jnp.int32),
        pltpu.VMEM((GRP, CHUNK), jnp.int32),
        pltpu.VMEM((GRP, CHUNK), jnp.int32),
        pltpu.VMEM((CHUNK, HID), jnp.float32),
        pltpu.VMEM((CHUNK, HID), jnp.float32),
        pltpu.VMEM((CHUNK, HID), jnp.float32),
        pltpu.VMEM((CHUNK, HID), jnp.float32),
        pltpu.VMEM((64, HID), jnp.float32),
        pltpu.VMEM_SHARED((ACC_ROWS, HID), jnp.float32),
        pltpu.SemaphoreType.DMA,
        pltpu.SemaphoreType.DMA,
        pltpu.SemaphoreType.DMA,
        pltpu.SemaphoreType.DMA,
        pltpu.SemaphoreType.DMA,
        pltpu.SemaphoreType.DMA,
        pltpu.SemaphoreType.DMA,
        pltpu.SemaphoreType.DMA,
    ],
)
def _sc_conv(hs_hbm, src_hbm, dst_hbm, acc_hbm,
             src_a, dst_a, src_b, dst_b, r0, r1, r2, r3, zbuf, acc_sh,
             g0, g1, g2, g3, s0, s1, s2, s3):
    c = lax.axis_index("c")
    s = lax.axis_index("s")
    wid = c * NS + s

    rows = (r0, r1, r2, r3)
    gsem = (g0, g1, g2, g3)
    ssem = (s0, s1, s2, s3)
    srcs = (src_a, src_b)
    dsts = (dst_a, dst_b)

    def fillz(i, _):
        for k in range(HID // 16):
            zbuf[i, pl.ds(k * 16, 16)] = jnp.zeros((16,), jnp.float32)
        return 0
    lax.fori_loop(0, 64, fillz, 0)

    base = s * ZROWS
    for k in range(ZROWS // 64):
        pltpu.async_copy(zbuf, acc_sh.at[pl.ds(base + k * 64, 64)], s0)
    pltpu.async_copy(zbuf.at[pl.ds(0, ZROWS % 64)],
                     acc_sh.at[pl.ds(base + (ZROWS // 64) * 64, ZROWS % 64)],
                     s1)
    for k in range(ZROWS // 64):
        pltpu.make_async_copy(zbuf, acc_sh.at[pl.ds(base, 64)], s0).wait()
    pltpu.make_async_copy(zbuf.at[pl.ds(0, ZROWS % 64)],
                          acc_sh.at[pl.ds(base, ZROWS % 64)], s1).wait()
    plsc.subcore_barrier()

    def stage(slot, g):
        pltpu.sync_copy(src_hbm.at[wid, g], srcs[slot])
        pltpu.sync_copy(dst_hbm.at[wid, g], dsts[slot])

    def gather_fire(slot, row, b):
        pltpu.async_copy(hs_hbm.at[srcs[slot].at[row]], rows[b], gsem[b])

    def gather_wait(slot, row, b):
        pltpu.make_async_copy(
            hs_hbm.at[srcs[slot].at[row]], rows[b], gsem[b]).wait()

    def scatter_fire(slot, row, b):
        pltpu.async_copy(rows[b], acc_sh.at[dsts[slot].at[row]], ssem[b],
                         add=True)

    def scatter_wait(b):
        pltpu.make_async_copy(rows[b], acc_sh.at[dsts[0].at[0]],
                              ssem[b]).wait()

    def pair_body(p, first):
        stage(0, 2 * p)
        for q in range(2 * GRP):
            slot, row = divmod(q, GRP)
            if q == GRP:
                stage(1, 2 * p + 1)
            b = q % NB
            if not (first and q < NB):
                scatter_wait(b)
            gather_fire(slot, row, b)
            qq = q - LAG
            if first and qq < 0:
                continue
            sslot, srow = divmod(qq % (2 * GRP), GRP)
            bb = qq % NB
            gather_wait(sslot, srow, bb)
            scatter_fire(sslot, srow, bb)

    pair_body(0, True)

    def body(p, _):
        pair_body(p, False)
        return 0
    lax.fori_loop(1, NGRP // 2, body, 0)

    for qq in range(2 * GRP - LAG, 2 * GRP):
        sslot, srow = divmod(qq, GRP)
        bb = qq % NB
        gather_wait(sslot, srow, bb)
        scatter_fire(sslot, srow, bb)
    for b in range(NB):
        scatter_wait(b)

    plsc.subcore_barrier()
    pltpu.sync_copy(acc_sh.at[pl.ds(s * ZROWS, ZROWS)],
                    acc_hbm.at[c, pl.ds(s * ZROWS, ZROWS)])


BM = 2000
GRID = N // BM


def _row_spec(w):
    return pl.BlockSpec((BM, w), lambda i: (i, 0))


def _full_spec(shape):
    return pl.BlockSpec(shape, lambda i: tuple(0 for _ in shape))


_acc_spec = pl.BlockSpec((NC, BM, HID), lambda i: (0, i, 0))
_deg_spec = pl.BlockSpec((NC, BM, HID), lambda i: (0, i, 0))


def _dinv(deg_block):
    deg = deg_block[0, :, 0:1] + deg_block[1, :, 0:1] + 1.0
    return lax.rsqrt(deg)


def _tc_enc_body(feat_ref, cond_ref, w1f_ref, w1c_ref, deg_ref, hs1_ref):
    dinv = _dinv(deg_ref[...])
    h = jnp.dot(feat_ref[...], w1f_ref[...], preferred_element_type=jnp.float32)
    h = h + jnp.dot(cond_ref[...], w1c_ref[...], preferred_element_type=jnp.float32)
    hs1_ref[...] = h * dinv


def _tc_mid1_body(acc_ref, hs1_ref, b1_ref, wml_ref, deg_ref, hs2_ref):
    dinv = _dinv(deg_ref[...])
    a = acc_ref[...]
    h = (a[0] + a[1] + hs1_ref[...]) * dinv + b1_ref[...]
    h = jnp.maximum(h, 0.0)
    hs2_ref[...] = jnp.dot(h, wml_ref[...],
                           preferred_element_type=jnp.float32) * dinv


def _tc_mid2_body(acc_ref, hs2_ref, bml_ref, noise_ref, cond_ref,
                  w4z_ref, w4c_ref, deg_ref, ml_ref, z_ref, hs3_ref):
    dinv = _dinv(deg_ref[...])
    a = acc_ref[...]
    ml = (a[0] + a[1] + hs2_ref[...]) * dinv + bml_ref[...]
    mean = ml[:, :LAT]
    logvar = ml[:, LAT:]
    z = noise_ref[...] * jnp.exp(0.5 * logvar) + mean
    h2 = jnp.dot(z, w4z_ref[...], preferred_element_type=jnp.float32)
    h2 = h2 + jnp.dot(cond_ref[...], w4c_ref[...],
                      preferred_element_type=jnp.float32)
    ml_ref[...] = ml
    z_ref[...] = z
    hs3_ref[...] = h2 * dinv


def _tc_mid3_body(acc_ref, hs3_ref, b4_ref, w5_ref, deg_ref, hs4_ref):
    dinv = _dinv(deg_ref[...])
    a = acc_ref[...]
    h2 = (a[0] + a[1] + hs3_ref[...]) * dinv + b4_ref[...]
    h2 = jnp.maximum(h2, 0.0)
    hs4_ref[...] = jnp.dot(h2, w5_ref[...],
                           preferred_element_type=jnp.float32) * dinv


def _tc_out_body(acc_ref, hs4_ref, b5_ref, deg_ref, out_ref):
    dinv = _dinv(deg_ref[...])
    a = acc_ref[...]
    out_ref[...] = (a[0] + a[1] + hs4_ref[...]) * dinv + b5_ref[...]


def _f32(shape):
    return jax.ShapeDtypeStruct(shape, jnp.float32)


def kernel(feature, condition, edge_index,
           W_enc_i2h, b_enc_i2h, W_mean, b_mean, W_logvar, b_logvar,
           W_dec_i2h, b_dec_i2h, W_dec_out, b_dec_out):
    src = edge_index[0].astype(jnp.int32)
    dst = edge_index[1].astype(jnp.int32)
    pad = E_PAD - E
    src_p = jnp.concatenate([src, jnp.zeros((pad,), jnp.int32)])
    dst_p = jnp.concatenate([dst, jnp.full((pad,), PAD_DST, jnp.int32)])
    src_p = src_p.reshape(NW, NGRP, GRP, CHUNK)
    dst_p = dst_p.reshape(NW, NGRP, GRP, CHUNK)

    degacc = _sc_degree(dst_p)
    deg = degacc[:, :N, :]

    W1f = W_enc_i2h[:FEAT]
    W1c = W_enc_i2h[FEAT:]
    hs1 = pl.pallas_call(
        _tc_enc_body,
        grid=(GRID,),
        in_specs=[_row_spec(FEAT), _row_spec(COND),
                  _full_spec((FEAT, HID)), _full_spec((COND, HID)), _deg_spec],
        out_specs=_row_spec(HID),
        out_shape=_f32((N, HID)),
    )(feature, condition, W1f, W1c, deg)

    acc1 = _sc_conv(hs1, src_p, dst_p)[:, :N]

    Wml = jnp.concatenate([W_mean, W_logvar], axis=1)
    b1 = b_enc_i2h.reshape(1, HID)
    hs2 = pl.pallas_call(
        _tc_mid1_body,
        grid=(GRID,),
        in_specs=[_acc_spec, _row_spec(HID), _full_spec((1, HID)),
                  _full_spec((HID, HID)), _deg_spec],
        out_specs=_row_spec(HID),
        out_shape=_f32((N, HID)),
    )(acc1, hs1, b1, Wml, deg)

    acc2 = _sc_conv(hs2, src_p, dst_p)[:, :N]

    noise = jax.random.normal(jax.random.key(42), (N, LAT), jnp.float32)
    b_ml = jnp.concatenate([b_mean, b_logvar]).reshape(1, HID)
    W4z = W_dec_i2h[:LAT]
    W4c = W_dec_i2h[LAT:]
    ml, z, hs3 = pl.pallas_call(
        _tc_mid2_body,
        grid=(GRID,),
        in_specs=[_acc_spec, _row_spec(HID), _full_spec((1, HID)),
                  _row_spec(LAT), _row_spec(COND),
                  _full_spec((LAT, HID)), _full_spec((COND, HID)), _deg_spec],
        out_specs=[_row_spec(HID), _row_spec(LAT), _row_spec(HID)],
        out_shape=[_f32((N, HID)), _f32((N, LAT)), _f32((N, HID))],
    )(acc2, hs2, b_ml, noise, condition, W4z, W4c, deg)

    acc3 = _sc_conv(hs3, src_p, dst_p)[:, :N]

    b4 = b_dec_i2h.reshape(1, HID)
    hs4 = pl.pallas_call(
        _tc_mid3_body,
        grid=(GRID,),
        in_specs=[_acc_spec, _row_spec(HID), _full_spec((1, HID)),
                  _full_spec((HID, FEAT)), _deg_spec],
        out_specs=_row_spec(FEAT),
        out_shape=_f32((N, FEAT)),
    )(acc3, hs3, b4, W_dec_out, deg)

    acc4 = _sc_conv(hs4, src_p, dst_p)[:, :N]

    b5 = b_dec_out.reshape(1, FEAT)
    out = pl.pallas_call(
        _tc_out_body,
        grid=(GRID,),
        in_specs=[_acc_spec, _row_spec(FEAT), _full_spec((1, FEAT)), _deg_spec],
        out_specs=_row_spec(FEAT),
        out_shape=_f32((N, FEAT)),
    )(acc4, hs4, b5, deg)

    mean = ml[:, :LAT]
    logvar = ml[:, LAT:]
    return (z, mean, logvar, out)


# speedup vs baseline: 22.8951x; 3.1171x over previous
import functools

import jax
import jax.numpy as jnp
from jax import lax
from jax.experimental import pallas as pl
from jax.experimental.pallas import tpu as pltpu
from jax.experimental.pallas import tpu_sc as plsc

N = 10000
E = 320000
FEAT = 128
COND = 16
HID = 128
LAT = 64

NC = 2
NS = 16
NW = NC * NS
CHUNK = 64
GRP = 4
NGRP = 40
NB = 4
LAG = 3
E_W = CHUNK * GRP * NGRP
E_PAD = E_W * NW
TOT = GRP * NGRP
PAD_DST = N
ACC_ROWS = 10112
ZROWS = ACC_ROWS // NS

_mesh = plsc.VectorSubcoreMesh(core_axis_name="c", subcore_axis_name="s")


@functools.partial(
    pl.kernel,
    out_type=jax.ShapeDtypeStruct((NC, ACC_ROWS, HID), jnp.float32),
    mesh=_mesh,
    scratch_types=[
        pltpu.VMEM((GRP, CHUNK), jnp.int32),
        pltpu.VMEM((CHUNK, HID), jnp.float32),
        pltpu.VMEM((64, HID), jnp.float32),
        pltpu.VMEM_SHARED((ACC_ROWS, HID), jnp.float32),
    ],
)
def _sc_degree(dst_hbm, deg_hbm, idx_g, ones_v, zbuf, deg_sh):
    c = lax.axis_index("c")
    s = lax.axis_index("s")
    wid = c * NS + s

    def fill(i, _):
        for k in range(HID // 16):
            ones_v[i, pl.ds(k * 16, 16)] = jnp.ones((16,), jnp.float32)
        return 0
    lax.fori_loop(0, CHUNK, fill, 0)

    def fillz(i, _):
        for k in range(HID // 16):
            zbuf[i, pl.ds(k * 16, 16)] = jnp.zeros((16,), jnp.float32)
        return 0
    lax.fori_loop(0, 64, fillz, 0)

    for k in range(ZROWS // 64):
        pltpu.sync_copy(zbuf, deg_sh.at[pl.ds(s * ZROWS + k * 64, 64)])
    pltpu.sync_copy(zbuf.at[pl.ds(0, ZROWS % 64)],
                    deg_sh.at[pl.ds(s * ZROWS + (ZROWS // 64) * 64,
                                    ZROWS % 64)])
    plsc.subcore_barrier()

    def body(g, _):
        pltpu.sync_copy(dst_hbm.at[wid, g], idx_g)
        for k in range(GRP):
            pltpu.sync_copy(ones_v, deg_sh.at[idx_g.at[k]], add=True)
        return 0
    lax.fori_loop(0, NGRP, body, 0)

    plsc.subcore_barrier()
    pltpu.sync_copy(deg_sh.at[pl.ds(s * ZROWS, ZROWS)],
                    deg_hbm.at[c, pl.ds(s * ZROWS, ZROWS)])


@functools.partial(
    pl.kernel,
    out_type=jax.ShapeDtypeStruct((NC, ACC_ROWS, HID), jnp.float32),
    mesh=_mesh,
    scratch_types=[
        pltpu.VMEM((GRP, CHUNK), jnp.int32),
        pltpu.VMEM((GRP, CHUNK), jnp.int32),
        pltpu.VMEM((GRP, CHUNK), jnp.int32),
        pltpu.VMEM((GRP, CHUNK), jnp.int32),
        pltpu.VMEM((CHUNK, HID), jnp.float32),
        pltpu.VMEM((CHUNK, HID), jnp.float32),
        pltpu.VMEM((CHUNK, HID), jnp.float32),
        pltpu.VMEM((CHUNK, HID), jnp.float32),
        pltpu.VMEM((64, HID), jnp.float32),
        pltpu.VMEM_SHARED((ACC_ROWS, HID), jnp.float32),
        pltpu.SemaphoreType.DMA,
        pltpu.SemaphoreType.DMA,
        pltpu.SemaphoreType.DMA,
        pltpu.SemaphoreType.DMA,
        pltpu.SemaphoreType.DMA,
        pltpu.SemaphoreType.DMA,
        pltpu.SemaphoreType.DMA,
        pltpu.SemaphoreType.DMA,
    ],
)
def _sc_conv(hs_hbm, src_hbm, dst_hbm, acc_hbm,
             src_a, dst_a, src_b, dst_b, r0, r1, r2, r3, zbuf, acc_sh,
             g0, g1, g2, g3, s0, s1, s2, s3):
    c = lax.axis_index("c")
    s = lax.axis_index("s")
    wid = c * NS + s

    rows = (r0, r1, r2, r3)
    gsem = (g0, g1, g2, g3)
    ssem = (s0, s1, s2, s3)
    srcs = (src_a, src_b)
    dsts = (dst_a, dst_b)

    def fillz(i, _):
        for k in range(HID // 16):
            zbuf[i, pl.ds(k * 16, 16)] = jnp.zeros((16,), jnp.float32)
        return 0
    lax.fori_loop(0, 64, fillz, 0)

    base = s * ZROWS
    for k in range(ZROWS // 64):
        pltpu.async_copy(zbuf, acc_sh.at[pl.ds(base + k * 64, 64)], s0)
    pltpu.async_copy(zbuf.at[pl.ds(0, ZROWS % 64)],
                     acc_sh.at[pl.ds(base + (ZROWS // 64) * 64, ZROWS % 64)],
                     s1)
    for k in range(ZROWS // 64):
        pltpu.make_async_copy(zbuf, acc_sh.at[pl.ds(base, 64)], s0).wait()
    pltpu.make_async_copy(zbuf.at[pl.ds(0, ZROWS % 64)],
                          acc_sh.at[pl.ds(base, ZROWS % 64)], s1).wait()
    plsc.subcore_barrier()

    def stage(slot, g):
        pltpu.sync_copy(src_hbm.at[wid, g], srcs[slot])
        pltpu.sync_copy(dst_hbm.at[wid, g], dsts[slot])

    def gather_fire(slot, row, b):
        pltpu.async_copy(hs_hbm.at[srcs[slot].at[row]], rows[b], gsem[b])

    def gather_wait(slot, row, b):
        pltpu.make_async_copy(
            hs_hbm.at[srcs[slot].at[row]], rows[b], gsem[b]).wait()

    def scatter_fire(slot, row, b):
        pltpu.async_copy(rows[b], acc_sh.at[dsts[slot].at[row]], ssem[b],
                         add=True)

    def scatter_wait(b):
        pltpu.make_async_copy(rows[b], acc_sh.at[dsts[0].at[0]],
                              ssem[b]).wait()

    def pair_body(p, first):
        stage(0, 2 * p)
        for q in range(2 * GRP):
            slot, row = divmod(q, GRP)
            if q == GRP:
                stage(1, 2 * p + 1)
            b = q % NB
            if not (first and q < NB):
                scatter_wait(b)
            gather_fire(slot, row, b)
            qq = q - LAG
            if first and qq < 0:
                continue
            sslot, srow = divmod(qq % (2 * GRP), GRP)
            bb = qq % NB
            gather_wait(sslot, srow, bb)
            scatter_fire(sslot, srow, bb)

    pair_body(0, True)

    def body(p, _):
        pair_body(p, False)
        return 0
    lax.fori_loop(1, NGRP // 2, body, 0)

    for qq in range(2 * GRP - LAG, 2 * GRP):
        sslot, srow = divmod(qq, GRP)
        bb = qq % NB
        gather_wait(sslot, srow, bb)
        scatter_fire(sslot, srow, bb)
    for b in range(NB):
        scatter_wait(b)

    plsc.subcore_barrier()
    pltpu.sync_copy(acc_sh.at[pl.ds(s * ZROWS, ZROWS)],
                    acc_hbm.at[c, pl.ds(s * ZROWS, ZROWS)])


BM = 2000
GRID = N // BM


def _row_spec(w):
    return pl.BlockSpec((BM, w), lambda i: (i, 0))


def _full_spec(shape):
    return pl.BlockSpec(shape, lambda i: tuple(0 for _ in shape))


_acc_spec = pl.BlockSpec((NC, BM, HID), lambda i: (0, i, 0))
_deg_spec = pl.BlockSpec((NC, BM, HID), lambda i: (0, i, 0))


def _dinv(deg_block):
    deg = deg_block[0, :, 0:1] + deg_block[1, :, 0:1] + 1.0
    return lax.rsqrt(deg)


def _tc_enc_body(feat_ref, cond_ref, w1f_ref, w1c_ref, deg_ref, hs1_ref):
    dinv = _dinv(deg_ref[...])
    h = jnp.dot(feat_ref[...], w1f_ref[...], preferred_element_type=jnp.float32)
    h = h + jnp.dot(cond_ref[...], w1c_ref[...], preferred_element_type=jnp.float32)
    hs1_ref[...] = h * dinv


def _tc_mid1_body(acc_ref, hs1_ref, b1_ref, wml_ref, deg_ref, hs2_ref):
    dinv = _dinv(deg_ref[...])
    a = acc_ref[...]
    h = (a[0] + a[1] + hs1_ref[...]) * dinv + b1_ref[...]
    h = jnp.maximum(h, 0.0)
    hs2_ref[...] = jnp.dot(h, wml_ref[...],
                           preferred_element_type=jnp.float32) * dinv


def _tc_mid2_body(acc_ref, hs2_ref, bml_ref, noise_ref, cond_ref,
                  w4z_ref, w4c_ref, deg_ref, ml_ref, z_ref, hs3_ref):
    dinv = _dinv(deg_ref[...])
    a = acc_ref[...]
    ml = (a[0] + a[1] + hs2_ref[...]) * dinv + bml_ref[...]
    mean = ml[:, :LAT]
    logvar = ml[:, LAT:]
    z = noise_ref[...] * jnp.exp(0.5 * logvar) + mean
    h2 = jnp.dot(z, w4z_ref[...], preferred_element_type=jnp.float32)
    h2 = h2 + jnp.dot(cond_ref[...], w4c_ref[...],
                      preferred_element_type=jnp.float32)
    ml_ref[...] = ml
    z_ref[...] = z
    hs3_ref[...] = h2 * dinv


def _tc_mid3_body(acc_ref, hs3_ref, b4_ref, w5_ref, deg_ref, hs4_ref):
    dinv = _dinv(deg_ref[...])
    a = acc_ref[...]
    h2 = (a[0] + a[1] + hs3_ref[...]) * dinv + b4_ref[...]
    h2 = jnp.maximum(h2, 0.0)
    hs4_ref[...] = jnp.dot(h2, w5_ref[...],
                           preferred_element_type=jnp.float32) * dinv


def _tc_out_body(acc_ref, hs4_ref, b5_ref, deg_ref, out_ref):
    dinv = _dinv(deg_ref[...])
    a = acc_ref[...]
    out_ref[...] = (a[0] + a[1] + hs4_ref[...]) * dinv + b5_ref[...]


def _f32(shape):
    return jax.ShapeDtypeStruct(shape, jnp.float32)


def kernel(feature, condition, edge_index,
           W_enc_i2h, b_enc_i2h, W_mean, b_mean, W_logvar, b_logvar,
           W_dec_i2h, b_dec_i2h, W_dec_out, b_dec_out):
    src = edge_index[0].astype(jnp.int32)
    dst = edge_index[1].astype(jnp.int32)
    pad = E_PAD - E
    src_p = jnp.concatenate([src, jnp.arange(pad, dtype=jnp.int32) % N])
    dst_p = jnp.concatenate([dst, jnp.full((pad,), PAD_DST, jnp.int32)])
    src_p = src_p.reshape(NW, NGRP, GRP, CHUNK)
    dst_p = dst_p.reshape(NW, NGRP, GRP, CHUNK)

    degacc = _sc_degree(dst_p)
    deg = degacc[:, :N, :]

    W1f = W_enc_i2h[:FEAT]
    W1c = W_enc_i2h[FEAT:]
    hs1 = pl.pallas_call(
        _tc_enc_body,
        grid=(GRID,),
        in_specs=[_row_spec(FEAT), _row_spec(COND),
                  _full_spec((FEAT, HID)), _full_spec((COND, HID)), _deg_spec],
        out_specs=_row_spec(HID),
        out_shape=_f32((N, HID)),
    )(feature, condition, W1f, W1c, deg)

    acc1 = _sc_conv(hs1, src_p, dst_p)[:, :N]

    Wml = jnp.concatenate([W_mean, W_logvar], axis=1)
    b1 = b_enc_i2h.reshape(1, HID)
    hs2 = pl.pallas_call(
        _tc_mid1_body,
        grid=(GRID,),
        in_specs=[_acc_spec, _row_spec(HID), _full_spec((1, HID)),
                  _full_spec((HID, HID)), _deg_spec],
        out_specs=_row_spec(HID),
        out_shape=_f32((N, HID)),
    )(acc1, hs1, b1, Wml, deg)

    acc2 = _sc_conv(hs2, src_p, dst_p)[:, :N]

    noise = jax.random.normal(jax.random.key(42), (N, LAT), jnp.float32)
    b_ml = jnp.concatenate([b_mean, b_logvar]).reshape(1, HID)
    W4z = W_dec_i2h[:LAT]
    W4c = W_dec_i2h[LAT:]
    ml, z, hs3 = pl.pallas_call(
        _tc_mid2_body,
        grid=(GRID,),
        in_specs=[_acc_spec, _row_spec(HID), _full_spec((1, HID)),
                  _row_spec(LAT), _row_spec(COND),
                  _full_spec((LAT, HID)), _full_spec((COND, HID)), _deg_spec],
        out_specs=[_row_spec(HID), _row_spec(LAT), _row_spec(HID)],
        out_shape=[_f32((N, HID)), _f32((N, LAT)), _f32((N, HID))],
    )(acc2, hs2, b_ml, noise, condition, W4z, W4c, deg)

    acc3 = _sc_conv(hs3, src_p, dst_p)[:, :N]

    b4 = b_dec_i2h.reshape(1, HID)
    hs4 = pl.pallas_call(
        _tc_mid3_body,
        grid=(GRID,),
        in_specs=[_acc_spec, _row_spec(HID), _full_spec((1, HID)),
                  _full_spec((HID, FEAT)), _deg_spec],
        out_specs=_row_spec(FEAT),
        out_shape=_f32((N, FEAT)),
    )(acc3, hs3, b4, W_dec_out, deg)

    acc4 = _sc_conv(hs4, src_p, dst_p)[:, :N]

    b5 = b_dec_out.reshape(1, FEAT)
    out = pl.pallas_call(
        _tc_out_body,
        grid=(GRID,),
        in_specs=[_acc_spec, _row_spec(FEAT), _full_spec((1, FEAT)), _deg_spec],
        out_specs=_row_spec(FEAT),
        out_shape=_f32((N, FEAT)),
    )(acc4, hs4, b5, deg)

    mean = ml[:, :LAT]
    logvar = ml[:, LAT:]
    return (z, mean, logvar, out)
